# baseline (speedup 1.0000x reference)
import functools

import jax
import jax.numpy as jnp
from jax import lax
from jax.experimental import pallas as pl
from jax.experimental.pallas import tpu as pltpu
from jax.experimental.pallas import tpu_sc as plsc

NC = 2
NS = 16
NW = NC * NS
D = 64
K = 128
NBUF = 4
EPS = 1e-5


def _rsqrt(a):
    i = lax.bitcast_convert_type(a, jnp.int32)
    y = lax.bitcast_convert_type(0x5F3759DF - (i >> 1), jnp.float32)
    for _ in range(3):
        y = y * (1.5 - 0.5 * a * y * y)
    return y


def _body(n_chunks, x_hbm, tab_hbm, gam_hbm, bet_hbm, out_hbm, *scratch):
    gam_v, bet_v = scratch[0], scratch[1]
    ibufs = scratch[2:2 + NBUF]
    rbufs = scratch[2 + NBUF:2 + 2 * NBUF]
    isems = scratch[2 + 2 * NBUF:2 + 3 * NBUF]
    gsems = scratch[2 + 3 * NBUF:2 + 4 * NBUF]
    osems = scratch[2 + 4 * NBUF:2 + 5 * NBUF]

    wid = lax.axis_index("s") * NC + lax.axis_index("c")
    base = wid * (n_chunks * K)

    pltpu.sync_copy(gam_hbm, gam_v)
    pltpu.sync_copy(bet_hbm, bet_v)
    gs = [gam_v[pl.ds(16 * j, 16)] for j in range(4)]
    bs = [bet_v[pl.ds(16 * j, 16)] for j in range(4)]

    def start_idx(c, s):
        pltpu.make_async_copy(
            x_hbm.at[pl.ds(base + c * K, K)], ibufs[s], isems[s]).start()

    def wait_idx(s):
        pltpu.make_async_copy(
            x_hbm.at[pl.ds(base, K)], ibufs[s], isems[s]).wait()

    def start_gather(s):
        pltpu.make_async_copy(tab_hbm.at[ibufs[s]], rbufs[s], gsems[s]).start()

    def wait_gather(s):
        pltpu.make_async_copy(tab_hbm.at[ibufs[s]], rbufs[s], gsems[s]).wait()

    def start_store(c, s):
        pltpu.make_async_copy(
            rbufs[s], out_hbm.at[pl.ds(base + c * K, K)], osems[s]).start()

    def wait_store(s):
        pltpu.make_async_copy(
            rbufs[s], out_hbm.at[pl.ds(base, K)], osems[s]).wait()

    iot = lax.iota(jnp.int32, 16)
    perms = [iot ^ t for t in (1, 2, 4, 8)]

    dnums = lax.GatherDimensionNumbers(
        offset_dims=(), collapsed_slice_dims=(0,), start_index_map=(0,))

    def bsum(v):
        for p in perms:
            v = v + lax.gather(v, p[:, None], dnums, slice_sizes=(1,),
                               mode=lax.GatherScatterMode.PROMISE_IN_BOUNDS)
        return v

    def ln_chunk(s):
        rbuf = rbufs[s]

        def row(r, carry):
            v = [rbuf[r, pl.ds(16 * j, 16)] for j in range(4)]
            tot = bsum((v[0] + v[1]) + (v[2] + v[3]))
            tot2 = bsum((v[0] * v[0] + v[1] * v[1])
                        + (v[2] * v[2] + v[3] * v[3]))
            mean = tot * (1.0 / D)
            var = tot2 * (1.0 / D) - mean * mean
            rstd = _rsqrt(var + EPS)
            for j in range(4):
                rbuf[r, pl.ds(16 * j, 16)] = (v[j] - mean) * rstd * gs[j] + bs[j]
            return carry

        lax.fori_loop(0, K, row, 0)

    for c in range(3):
        start_idx(c, c)
    wait_idx(0)
    start_gather(0)

    def outer(i, carry):
        c0 = i * NBUF
        for h in range(NBUF):
            s = h
            c = c0 + h
            wait_gather(s)

            @pl.when(c + 3 < n_chunks)
            def _():
                start_idx(c + 3, (h + 3) % NBUF)

            @pl.when(c + 1 < n_chunks)
            def _():
                s1 = (h + 1) % NBUF

                @pl.when(c - 3 >= 0)
                def _():
                    wait_store(s1)

                wait_idx(s1)
                start_gather(s1)

            ln_chunk(s)
            start_store(c, s)
        return carry

    lax.fori_loop(0, n_chunks // NBUF, outer, 0)
    for s in range(NBUF):
        wait_store(s)


def kernel(x, tok_embed, ln_gamma, ln_beta):
    b, sq = x.shape
    n = b * sq
    assert n % (NW * K) == 0
    n_chunks = n // (NW * K)

    mesh = plsc.VectorSubcoreMesh(core_axis_name="c", subcore_axis_name="s")
    fn = pl.kernel(
        functools.partial(_body, n_chunks),
        out_type=jax.ShapeDtypeStruct((n, D), jnp.float32),
        mesh=mesh,
        compiler_params=pltpu.CompilerParams(use_tc_tiling_on_sc=False),
        scratch_types=(
            [pltpu.VMEM((D,), jnp.float32),
             pltpu.VMEM((D,), jnp.float32)]
            + [pltpu.VMEM((K,), jnp.int32) for _ in range(NBUF)]
            + [pltpu.VMEM((K, D), jnp.float32) for _ in range(NBUF)]
            + [pltpu.SemaphoreType.DMA for _ in range(3 * NBUF)]
        ),
    )
    out = fn(x.reshape(n), tok_embed, ln_gamma, ln_beta)
    return out.reshape(b, sq, D)


# speedup vs baseline: 1.4463x; 1.4463x over previous
import functools

import jax
import jax.numpy as jnp
from jax import lax
from jax.experimental import pallas as pl
from jax.experimental.pallas import tpu as pltpu
from jax.experimental.pallas import tpu_sc as plsc

NC = 2
NS = 16
NW = NC * NS
D = 64
K = 200
KA, KB = 104, 96
NBUF = 4
UNROLL = 8
EPS = 1e-5


def _body(n_chunks, x_hbm, tab_hbm, gam_hbm, bet_hbm, out_hbm, *scratch):
    gam_v, bet_v = scratch[0], scratch[1]
    ibufs = scratch[2:2 + NBUF]
    rbufs = scratch[2 + NBUF:2 + 2 * NBUF]
    isems = scratch[2 + 2 * NBUF:2 + 3 * NBUF]
    gsems = scratch[2 + 3 * NBUF:2 + 4 * NBUF]
    osems = scratch[2 + 4 * NBUF:2 + 5 * NBUF]

    wid = lax.axis_index("s") * NC + lax.axis_index("c")
    base = wid * n_chunks

    pltpu.sync_copy(gam_hbm, gam_v)
    pltpu.sync_copy(bet_hbm, bet_v)
    gs = [gam_v[pl.ds(16 * j, 16)] for j in range(4)]
    bs = [bet_v[pl.ds(16 * j, 16)] for j in range(4)]

    def start_idx(c, s):
        pltpu.make_async_copy(x_hbm.at[base + c], ibufs[s], isems[s]).start()

    def wait_idx(s):
        pltpu.make_async_copy(x_hbm.at[base], ibufs[s], isems[s]).wait()

    def gather_parts(s):
        return (
            pltpu.make_async_copy(
                tab_hbm.at[ibufs[s].at[pl.ds(0, KA)]],
                rbufs[s].at[pl.ds(0, KA)], gsems[s]),
            pltpu.make_async_copy(
                tab_hbm.at[ibufs[s].at[pl.ds(KA, KB)]],
                rbufs[s].at[pl.ds(KA, KB)], gsems[s]),
        )

    def start_gather(s):
        for p in gather_parts(s):
            p.start()

    def wait_gather(s):
        for p in gather_parts(s):
            p.wait()

    def start_store(c, s):
        pltpu.make_async_copy(rbufs[s], out_hbm.at[base + c], osems[s]).start()

    def wait_store(s):
        pltpu.make_async_copy(rbufs[s], out_hbm.at[base], osems[s]).wait()

    iot = lax.iota(jnp.int32, 16)
    perms = [(iot ^ t)[:, None] for t in (1, 2, 4, 8)]
    dnums = lax.GatherDimensionNumbers(
        offset_dims=(), collapsed_slice_dims=(0,), start_index_map=(0,))

    def bsum(v):
        for p in perms:
            v = v + lax.gather(v, p, dnums, slice_sizes=(1,),
                               mode=lax.GatherScatterMode.PROMISE_IN_BOUNDS)
        return v

    def ln_rows(rbuf, r):
        for u in range(UNROLL):
            v = [rbuf[r + u, pl.ds(16 * j, 16)] for j in range(4)]
            tot = bsum((v[0] + v[1]) + (v[2] + v[3]))
            tot2 = bsum((v[0] * v[0] + v[1] * v[1])
                        + (v[2] * v[2] + v[3] * v[3]))
            mean = tot * (1.0 / D)
            var = tot2 * (1.0 / D) - mean * mean
            a = var + EPS
            i = lax.bitcast_convert_type(a, jnp.int32)
            y = lax.bitcast_convert_type(0x5F3759DF - (i >> 1), jnp.float32)
            y = y * (1.5 - 0.5 * a * y * y)
            y = y * (1.5 - 0.5 * a * y * y)
            y = y * (1.5 - 0.5 * a * y * y)
            for j in range(4):
                rbuf[r + u, pl.ds(16 * j, 16)] = \
                    (v[j] - mean) * (y * gs[j]) + bs[j]

    def ln_chunk(s):
        rbuf = rbufs[s]

        def rowblk(rb, carry):
            ln_rows(rbuf, rb * UNROLL)
            return carry

        lax.fori_loop(0, K // UNROLL, rowblk, 0)

    for c in range(3):
        start_idx(c, c)
    wait_idx(0)
    start_gather(0)

    def outer(i, carry):
        c0 = i * NBUF
        for h in range(NBUF):
            s = h
            c = c0 + h
            wait_gather(s)

            @pl.when(c + 3 < n_chunks)
            def _():
                start_idx(c + 3, (h + 3) % NBUF)

            @pl.when(c + 1 < n_chunks)
            def _():
                s1 = (h + 1) % NBUF

                @pl.when(c - 3 >= 0)
                def _():
                    wait_store(s1)

                wait_idx(s1)
                start_gather(s1)

            ln_chunk(s)
            start_store(c, s)
        return carry

    lax.fori_loop(0, n_chunks // NBUF, outer, 0)
    for s in range(NBUF):
        wait_store(s)


def kernel(x, tok_embed, ln_gamma, ln_beta):
    b, sq = x.shape
    assert sq == K and b % (NW * NBUF) == 0
    n_chunks = b // NW

    mesh = plsc.VectorSubcoreMesh(core_axis_name="c", subcore_axis_name="s")
    fn = pl.kernel(
        functools.partial(_body, n_chunks),
        out_type=jax.ShapeDtypeStruct((b, sq, D), jnp.float32),
        mesh=mesh,
        compiler_params=pltpu.CompilerParams(use_tc_tiling_on_sc=False),
        scratch_types=(
            [pltpu.VMEM((D,), jnp.float32),
             pltpu.VMEM((D,), jnp.float32)]
            + [pltpu.VMEM((K,), jnp.int32) for _ in range(NBUF)]
            + [pltpu.VMEM((K, D), jnp.float32) for _ in range(NBUF)]
            + [pltpu.SemaphoreType.DMA for _ in range(3 * NBUF)]
        ),
    )
    return fn(x, tok_embed, ln_gamma, ln_beta)
